# baseline (speedup 1.0000x reference)
import numpy as np
import jax
import jax.numpy as jnp
from jax import lax
from jax.experimental import pallas as pl
from jax.experimental.pallas import tpu as pltpu
from jax.experimental.pallas import tpu_sc as plsc

_VOCAB = 1000000
_EMBED = 64
_BATCH = 4096
_SEQLEN = 200

_NC = 2
_NS = 16
_NW = _NC * _NS

_CROWS = 40
_CPS = _SEQLEN // _CROWS
_SPW = _BATCH // _NW
_CPW = _CPS * _SPW
_NBUF = 10


def _positional_np(seq_len, d_model):
    position = np.arange(seq_len)[:, None].astype(np.float32)
    div_term = np.exp(
        np.arange(0, d_model, 2).astype(np.float32) * -(np.log(10000.0) / d_model)
    )
    pe = np.zeros((seq_len, d_model), dtype=np.float32)
    pe[:, 0::2] = np.sin(position * div_term)
    pe[:, 1::2] = np.cos(position * div_term)
    return pe


def _sc_body(seq_hbm, pe_hbm, table_hbm, out_hbm, idx_v, pe_v, bufs, gsems, osems):
    wid = lax.axis_index("s") * _NC + lax.axis_index("c")
    sbase = wid * _SPW

    pltpu.sync_copy(seq_hbm.at[pl.ds(sbase, _SPW)], idx_v)
    pltpu.sync_copy(pe_hbm, pe_v)

    def add_pe(h, buf):
        @pl.loop(0, _CROWS, unroll=4)
        def _row(r):
            for k in range(_EMBED // 16):
                sl = pl.ds(k * 16, 16)
                buf[r, sl] = buf[r, sl] + pe_v[h * _CROWS + r, sl]

    def gather_start(c, h, b):
        idx = idx_v.at[lax.div(c, _CPS), pl.ds(h * _CROWS, _CROWS)]
        pltpu.async_copy(table_hbm.at[idx], bufs[b], gsems[b])

    def gather_wait(c, h, b):
        idx = idx_v.at[lax.div(c, _CPS), pl.ds(h * _CROWS, _CROWS)]
        pltpu.make_async_copy(table_hbm.at[idx], bufs[b], gsems[b]).wait()

    def out_ref(c, h):
        return out_hbm.at[sbase + lax.div(c, _CPS), pl.ds(h * _CROWS, _CROWS), :]

    def out_start(c, h, b):
        pltpu.async_copy(bufs[b], out_ref(c, h), osems[b])

    def out_wait(c, h, b):
        pltpu.make_async_copy(bufs[b], out_ref(c, h), osems[b]).wait()

    for c in range(_NBUF - 1):
        gather_start(c, c % _CPS, c)

    @pl.loop(0, _CPW, step=_NBUF)
    def _steady(g):
        for j in range(_NBUF):
            c = g + j
            h = j % _CPS
            nb = (j + _NBUF - 1) % _NBUF
            nh = (j + _NBUF - 1) % _CPS
            gather_wait(c, h, j)

            @pl.when(c + _NBUF - 1 < _CPW)
            def _prefetch(c=c, nb=nb, nh=nh):
                @pl.when(c >= 1)
                def _drain_prev():
                    out_wait(c - 1, nh, nb)

                gather_start(c + _NBUF - 1, nh, nb)

            add_pe(h, bufs[j])
            out_start(c, h, j)

    for c in range(_CPW - _NBUF, _CPW):
        out_wait(c, c % _CPS, c % _NBUF)


@jax.jit
def kernel(sequence, token_table):
    pe = jnp.asarray(_positional_np(_SEQLEN, _EMBED))

    mesh = plsc.VectorSubcoreMesh(
        core_axis_name="c", subcore_axis_name="s", num_cores=_NC, num_subcores=_NS
    )
    run = pl.kernel(
        _sc_body,
        out_type=jax.ShapeDtypeStruct((_BATCH, _SEQLEN, _EMBED), jnp.float32),
        mesh=mesh,
        scratch_types=[
            pltpu.VMEM((_SPW, _SEQLEN), jnp.int32),
            pltpu.VMEM((_SEQLEN, _EMBED), jnp.float32),
            tuple(pltpu.VMEM((_CROWS, _EMBED), jnp.float32) for _ in range(_NBUF)),
            tuple(pltpu.SemaphoreType.DMA for _ in range(_NBUF)),
            tuple(pltpu.SemaphoreType.DMA for _ in range(_NBUF)),
        ],
        compiler_params=pltpu.CompilerParams(use_tc_tiling_on_sc=False),
    )
    return run(sequence, pe, token_table)


# speedup vs baseline: 1.0000x; 1.0000x over previous
import numpy as np
import jax
import jax.numpy as jnp
from jax import lax
from jax.experimental import pallas as pl
from jax.experimental.layout import Format, Layout
from jax.experimental.pallas import tpu as pltpu
from jax.experimental.pallas import tpu_sc as plsc

_VOCAB = 1000000
_EMBED = 64
_BATCH = 4096
_SEQLEN = 200

_NC = 2
_NS = 16
_NW = _NC * _NS

_CROWS = 40
_CPS = _SEQLEN // _CROWS
_SPW = _BATCH // _NW
_CPW = _CPS * _SPW
_NBUF = 10


def _positional_np(seq_len, d_model):
    position = np.arange(seq_len)[:, None].astype(np.float32)
    div_term = np.exp(
        np.arange(0, d_model, 2).astype(np.float32) * -(np.log(10000.0) / d_model)
    )
    pe = np.zeros((seq_len, d_model), dtype=np.float32)
    pe[:, 0::2] = np.sin(position * div_term)
    pe[:, 1::2] = np.cos(position * div_term)
    return pe


def _sc_body(seq_hbm, pe_hbm, table_hbm, out_hbm, idx_v, pe_v, bufs, gsems, osems):
    wid = lax.axis_index("s") * _NC + lax.axis_index("c")
    sbase = wid * _SPW

    pltpu.sync_copy(seq_hbm.at[pl.ds(sbase, _SPW)], idx_v)
    pltpu.sync_copy(pe_hbm, pe_v)

    def add_pe(h, buf):
        @pl.loop(0, _CROWS, unroll=4)
        def _row(r):
            for k in range(_EMBED // 16):
                sl = pl.ds(k * 16, 16)
                buf[r, sl] = buf[r, sl] + pe_v[h * _CROWS + r, sl]

    def gather_start(c, h, b):
        idx = idx_v.at[lax.div(c, _CPS), pl.ds(h * _CROWS, _CROWS)]
        pltpu.async_copy(table_hbm.at[idx], bufs[b], gsems[b])

    def gather_wait(c, h, b):
        idx = idx_v.at[lax.div(c, _CPS), pl.ds(h * _CROWS, _CROWS)]
        pltpu.make_async_copy(table_hbm.at[idx], bufs[b], gsems[b]).wait()

    def out_ref(c, h):
        return out_hbm.at[sbase + lax.div(c, _CPS), pl.ds(h * _CROWS, _CROWS), :]

    def out_start(c, h, b):
        pltpu.async_copy(bufs[b], out_ref(c, h), osems[b])

    def out_wait(c, h, b):
        pltpu.make_async_copy(bufs[b], out_ref(c, h), osems[b]).wait()

    for c in range(_NBUF - 1):
        gather_start(c, c % _CPS, c)

    @pl.loop(0, _CPW, step=_NBUF)
    def _steady(g):
        for j in range(_NBUF):
            c = g + j
            h = j % _CPS
            nb = (j + _NBUF - 1) % _NBUF
            nh = (j + _NBUF - 1) % _CPS
            gather_wait(c, h, j)

            @pl.when(c + _NBUF - 1 < _CPW)
            def _prefetch(c=c, nb=nb, nh=nh):
                @pl.when(c >= 1)
                def _drain_prev():
                    out_wait(c - 1, nh, nb)

                gather_start(c + _NBUF - 1, nh, nb)

            add_pe(h, bufs[j])
            out_start(c, h, j)

    for c in range(_CPW - _NBUF, _CPW):
        out_wait(c, c % _CPS, c % _NBUF)


def _kernel_impl(sequence, token_table):
    pe = jnp.asarray(_positional_np(_SEQLEN, _EMBED))

    mesh = plsc.VectorSubcoreMesh(
        core_axis_name="c", subcore_axis_name="s", num_cores=_NC, num_subcores=_NS
    )
    run = pl.kernel(
        _sc_body,
        out_type=jax.ShapeDtypeStruct((_BATCH, _SEQLEN, _EMBED), jnp.float32),
        mesh=mesh,
        scratch_types=[
            pltpu.VMEM((_SPW, _SEQLEN), jnp.int32),
            pltpu.VMEM((_SEQLEN, _EMBED), jnp.float32),
            tuple(pltpu.VMEM((_CROWS, _EMBED), jnp.float32) for _ in range(_NBUF)),
            tuple(pltpu.SemaphoreType.DMA for _ in range(_NBUF)),
            tuple(pltpu.SemaphoreType.DMA for _ in range(_NBUF)),
        ],
        compiler_params=pltpu.CompilerParams(use_tc_tiling_on_sc=False),
    )
    return run(sequence, pe, token_table)


_JITTED = None


def kernel(sequence, token_table):
    global _JITTED
    if _JITTED is None:
        dev = getattr(sequence, "sharding", None)
        if not isinstance(dev, jax.sharding.SingleDeviceSharding):
            dev = jax.sharding.SingleDeviceSharding(jax.devices()[0])
        _JITTED = jax.jit(
            _kernel_impl,
            in_shardings=(
                Format(Layout(major_to_minor=(0, 1), tiling=((8, 128),)), dev),
                Format(Layout(major_to_minor=(0, 1), tiling=((8, 128),)), dev),
            ),
            out_shardings=Format(
                Layout(major_to_minor=(0, 1, 2), tiling=((8,),)), dev
            ),
        )
    return _JITTED(sequence, token_table)


# speedup vs baseline: 1.3449x; 1.3449x over previous
import numpy as np
import jax
import jax.numpy as jnp
from jax import lax
from jax.experimental import pallas as pl
from jax.experimental.layout import Format, Layout, with_layout_constraint
from jax.experimental.pallas import tpu as pltpu
from jax.experimental.pallas import tpu_sc as plsc

_VOCAB = 1000000
_EMBED = 64
_BATCH = 4096
_SEQLEN = 200

_NC = 2
_NS = 16
_NW = _NC * _NS

_CROWS = 40
_CPS = _SEQLEN // _CROWS
_SPW = _BATCH // _NW
_CPW = _CPS * _SPW
_NBUF = 10


def _positional_np(seq_len, d_model):
    position = np.arange(seq_len)[:, None].astype(np.float32)
    div_term = np.exp(
        np.arange(0, d_model, 2).astype(np.float32) * -(np.log(10000.0) / d_model)
    )
    pe = np.zeros((seq_len, d_model), dtype=np.float32)
    pe[:, 0::2] = np.sin(position * div_term)
    pe[:, 1::2] = np.cos(position * div_term)
    return pe


def _sc_body(seq_hbm, pe_hbm, table_hbm, out_hbm, idx_v, pe_v, bufs, gsems, osems):
    wid = lax.axis_index("s") * _NC + lax.axis_index("c")
    sbase = wid * _SPW

    pltpu.sync_copy(seq_hbm.at[pl.ds(sbase, _SPW)], idx_v)
    pltpu.sync_copy(pe_hbm, pe_v)

    def add_pe(h, buf):
        @pl.loop(0, _CROWS, unroll=4)
        def _row(r):
            for k in range(_EMBED // 16):
                sl = pl.ds(k * 16, 16)
                buf[r, sl] = buf[r, sl] + pe_v[h * _CROWS + r, sl]

    def gather_start(c, h, b):
        idx = idx_v.at[lax.div(c, _CPS), pl.ds(h * _CROWS, _CROWS)]
        pltpu.async_copy(table_hbm.at[idx], bufs[b], gsems[b])

    def gather_wait(c, h, b):
        idx = idx_v.at[lax.div(c, _CPS), pl.ds(h * _CROWS, _CROWS)]
        pltpu.make_async_copy(table_hbm.at[idx], bufs[b], gsems[b]).wait()

    def out_ref(c, h):
        return out_hbm.at[sbase + lax.div(c, _CPS), pl.ds(h * _CROWS, _CROWS), :]

    def out_start(c, h, b):
        pltpu.async_copy(bufs[b], out_ref(c, h), osems[b])

    def out_wait(c, h, b):
        pltpu.make_async_copy(bufs[b], out_ref(c, h), osems[b]).wait()

    for c in range(_NBUF - 1):
        gather_start(c, c % _CPS, c)

    @pl.loop(0, _CPW, step=_NBUF)
    def _steady(g):
        for j in range(_NBUF):
            c = g + j
            h = j % _CPS
            nb = (j + _NBUF - 1) % _NBUF
            nh = (j + _NBUF - 1) % _CPS
            gather_wait(c, h, j)

            @pl.when(c + _NBUF - 1 < _CPW)
            def _prefetch(c=c, nb=nb, nh=nh):
                @pl.when(c >= 1)
                def _drain_prev():
                    out_wait(c - 1, nh, nb)

                gather_start(c + _NBUF - 1, nh, nb)

            add_pe(h, bufs[j])
            out_start(c, h, j)

    for c in range(_CPW - _NBUF, _CPW):
        out_wait(c, c % _CPS, c % _NBUF)


def _kernel_impl(sequence, token_table):
    pe = jnp.asarray(_positional_np(_SEQLEN, _EMBED))

    mesh = plsc.VectorSubcoreMesh(
        core_axis_name="c", subcore_axis_name="s", num_cores=_NC, num_subcores=_NS
    )
    run = pl.kernel(
        _sc_body,
        out_type=jax.ShapeDtypeStruct((_BATCH, _SEQLEN, _EMBED), jnp.float32),
        mesh=mesh,
        scratch_types=[
            pltpu.VMEM((_SPW, _SEQLEN), jnp.int32),
            pltpu.VMEM((_SEQLEN, _EMBED), jnp.float32),
            tuple(pltpu.VMEM((_CROWS, _EMBED), jnp.float32) for _ in range(_NBUF)),
            tuple(pltpu.SemaphoreType.DMA for _ in range(_NBUF)),
            tuple(pltpu.SemaphoreType.DMA for _ in range(_NBUF)),
        ],
        compiler_params=pltpu.CompilerParams(use_tc_tiling_on_sc=False),
    )
    return run(sequence, pe, token_table)


def kernel(sequence, token_table):
    sequence = with_layout_constraint(
        sequence, Layout(major_to_minor=(0, 1), tiling=((8, 128),))
    )
    token_table = with_layout_constraint(
        token_table, Layout(major_to_minor=(0, 1), tiling=((8, 128),))
    )
    out = _kernel_impl(sequence, token_table)
    return with_layout_constraint(
        out, Layout(major_to_minor=(0, 1, 2), tiling=((8,),))
    )


# speedup vs baseline: 1.4357x; 1.0675x over previous
import numpy as np
import jax
import jax.numpy as jnp
from jax import lax
from jax.experimental import pallas as pl
from jax.experimental.layout import Format, Layout, with_layout_constraint
from jax.experimental.pallas import tpu as pltpu
from jax.experimental.pallas import tpu_sc as plsc

_VOCAB = 1000000
_EMBED = 64
_BATCH = 4096
_SEQLEN = 200

_NC = 2
_NS = 16
_NW = _NC * _NS

_CROWS = 40
_CPS = _SEQLEN // _CROWS
_SPW = _BATCH // _NW
_CPW = _CPS * _SPW
_NBUF = 10


def _positional_np(seq_len, d_model):
    position = np.arange(seq_len)[:, None].astype(np.float32)
    div_term = np.exp(
        np.arange(0, d_model, 2).astype(np.float32) * -(np.log(10000.0) / d_model)
    )
    pe = np.zeros((seq_len, d_model), dtype=np.float32)
    pe[:, 0::2] = np.sin(position * div_term)
    pe[:, 1::2] = np.cos(position * div_term)
    return pe


def _sc_body(seq_hbm, pe_hbm, table_hbm, out_hbm, idx_v, pe_v, bufs, gsems, osems):
    wid = lax.axis_index("s") * _NC + lax.axis_index("c")
    sbase = wid * _SPW

    pltpu.sync_copy(seq_hbm.at[pl.ds(sbase, _SPW)], idx_v)
    pltpu.sync_copy(pe_hbm, pe_v)

    def add_pe(h, buf):
        @pl.loop(0, _CROWS, unroll=8)
        def _row(r):
            for k in range(_EMBED // 16):
                sl = pl.ds(k * 16, 16)
                plsc.addupdate(buf.at[r, sl], pe_v[h * _CROWS + r, sl])

    def gather_start(c, h, b):
        idx = idx_v.at[lax.div(c, _CPS), pl.ds(h * _CROWS, _CROWS)]
        pltpu.async_copy(table_hbm.at[idx], bufs[b], gsems[b])

    def gather_wait(c, h, b):
        idx = idx_v.at[lax.div(c, _CPS), pl.ds(h * _CROWS, _CROWS)]
        pltpu.make_async_copy(table_hbm.at[idx], bufs[b], gsems[b]).wait()

    def out_ref(c, h):
        return out_hbm.at[sbase + lax.div(c, _CPS), pl.ds(h * _CROWS, _CROWS), :]

    def out_start(c, h, b):
        pltpu.async_copy(bufs[b], out_ref(c, h), osems[b])

    def out_wait(c, h, b):
        pltpu.make_async_copy(bufs[b], out_ref(c, h), osems[b]).wait()

    for c in range(_NBUF - 1):
        gather_start(c, c % _CPS, c)

    @pl.loop(0, _CPW, step=_NBUF)
    def _steady(g):
        for j in range(_NBUF):
            c = g + j
            h = j % _CPS
            nb = (j + _NBUF - 1) % _NBUF
            nh = (j + _NBUF - 1) % _CPS
            gather_wait(c, h, j)

            @pl.when(c + _NBUF - 1 < _CPW)
            def _prefetch(c=c, nb=nb, nh=nh):
                @pl.when(c >= 1)
                def _drain_prev():
                    out_wait(c - 1, nh, nb)

                gather_start(c + _NBUF - 1, nh, nb)

            add_pe(h, bufs[j])
            out_start(c, h, j)

    for c in range(_CPW - _NBUF, _CPW):
        out_wait(c, c % _CPS, c % _NBUF)


def _kernel_impl(sequence, token_table):
    pe = jnp.asarray(_positional_np(_SEQLEN, _EMBED))

    mesh = plsc.VectorSubcoreMesh(
        core_axis_name="c", subcore_axis_name="s", num_cores=_NC, num_subcores=_NS
    )
    run = pl.kernel(
        _sc_body,
        out_type=jax.ShapeDtypeStruct((_BATCH, _SEQLEN, _EMBED), jnp.float32),
        mesh=mesh,
        scratch_types=[
            pltpu.VMEM((_SPW, _SEQLEN), jnp.int32),
            pltpu.VMEM((_SEQLEN, _EMBED), jnp.float32),
            tuple(pltpu.VMEM((_CROWS, _EMBED), jnp.float32) for _ in range(_NBUF)),
            tuple(pltpu.SemaphoreType.DMA for _ in range(_NBUF)),
            tuple(pltpu.SemaphoreType.DMA for _ in range(_NBUF)),
        ],
        compiler_params=pltpu.CompilerParams(use_tc_tiling_on_sc=False),
    )
    return run(sequence, pe, token_table)


def kernel(sequence, token_table):
    sequence = with_layout_constraint(
        sequence, Layout(major_to_minor=(0, 1), tiling=((8, 128),))
    )
    token_table = with_layout_constraint(
        token_table, Layout(major_to_minor=(0, 1), tiling=((8, 128),))
    )
    return _kernel_impl(sequence, token_table)
